# baseline (speedup 1.0000x reference)
import jax
import jax.numpy as jnp
from jax import lax
from jax.experimental import pallas as pl
from jax.experimental.pallas import tpu as pltpu
from jax.experimental.pallas import tpu_sc as plsc

R = 128
N = 10000
E = 320000
NC = 2
NS = 16
L = 16
RB = 8
EH = E // NC
CH = 1280
NCH = EH // CH

_HI = -65536


def _body(mp_hbm, idx_hbm, out_hbm,
          rows, ipkA, ipkB, obA, obB,
          isemA, isemB, osemA, osemB):
    grp = lax.axis_index("s")
    half = lax.axis_index("c")
    r0 = pl.multiple_of(grp * RB, 8)
    ebase = pl.multiple_of(half * EH, 128)

    for r in range(RB):
        pltpu.sync_copy(mp_hbm.at[pl.ds((r0 + r) * N, N)],
                        rows.at[pl.ds(r * N, N)])

    def compute_chunk(ipk, ob):
        @plsc.parallel_loop(0, CH, step=L, unroll=1)
        def vec_body(s):
            iv = ipk[pl.ds(s, L)]
            ia = iv & 0xFFFF
            ib = lax.shift_right_logical(iv, 16)
            for r in range(RB):
                row = rows.at[pl.ds(r * N, N)]
                w1 = plsc.load_gather(row, [ia])
                w2 = plsc.load_gather(row, [ib])
                m1 = plsc.bitcast(w1, jnp.float32)
                p1 = plsc.bitcast(lax.shift_left(w1, 16), jnp.float32)
                m2 = plsc.bitcast(w2, jnp.float32)
                p2 = plsc.bitcast(lax.shift_left(w2, 16), jnp.float32)
                t1 = m1 * p2
                t2 = m2 * p1
                ob[r, pl.ds(s, L)] = (t1 * t2) / (t1 + t2)

    def phase(k, ipkX, isemX, obX, osemX, ipkY, isemY, pre_c, pre_ok):
        @pl.when(pre_ok)
        def _():
            pltpu.async_copy(idx_hbm.at[pl.ds(ebase + pre_c * CH, CH)],
                             ipkY, isemY)

        pltpu.make_async_copy(idx_hbm.at[pl.ds(0, CH)], ipkX, isemX).wait()

        @pl.when(k >= 2)
        def _():
            pltpu.make_async_copy(
                obX, out_hbm.at[pl.ds(0, RB), pl.ds(0, CH)], osemX).wait()

        compute_chunk(ipkX, obX)

        pltpu.async_copy(
            obX,
            out_hbm.at[pl.ds(r0, RB), pl.ds(ebase + k * CH, CH)], osemX)

    pltpu.async_copy(idx_hbm.at[pl.ds(ebase, CH)], ipkA, isemA)

    def pair_body(j, carry):
        k = 2 * j
        phase(k, ipkA, isemA, obA, osemA, ipkB, isemB, k + 1, k + 1 < NCH)
        phase(k + 1, ipkB, isemB, obB, osemB, ipkA, isemA, k + 2, k + 2 < NCH)
        return carry

    lax.fori_loop(0, NCH // 2, pair_body, 0)
    phase(NCH - 1, ipkA, isemA, obA, osemA, ipkB, isemB, 0, False)

    pltpu.make_async_copy(
        obA, out_hbm.at[pl.ds(0, RB), pl.ds(0, CH)], osemA).wait()
    pltpu.make_async_copy(
        obB, out_hbm.at[pl.ds(0, RB), pl.ds(0, CH)], osemB).wait()


def kernel(m, polar, indices):
    mb = lax.bitcast_convert_type(
        m.astype(jnp.bfloat16), jnp.uint16).astype(jnp.uint32)
    pb = lax.bitcast_convert_type(
        polar.astype(jnp.bfloat16), jnp.uint16).astype(jnp.uint32)
    mp = lax.bitcast_convert_type((mb << 16) | pb, jnp.int32).reshape(-1)
    ipk = indices[0] | (indices[1] << 16)

    mesh = plsc.VectorSubcoreMesh(core_axis_name="c", subcore_axis_name="s")
    f = pl.kernel(
        _body,
        out_type=jax.ShapeDtypeStruct((R, E), jnp.float32),
        mesh=mesh,
        compiler_params=pltpu.CompilerParams(needs_layout_passes=False),
        scratch_types=[
            pltpu.VMEM((RB * N,), jnp.int32),
            pltpu.VMEM((CH,), jnp.int32),
            pltpu.VMEM((CH,), jnp.int32),
            pltpu.VMEM((RB, CH), jnp.float32),
            pltpu.VMEM((RB, CH), jnp.float32),
            pltpu.SemaphoreType.DMA,
            pltpu.SemaphoreType.DMA,
            pltpu.SemaphoreType.DMA,
            pltpu.SemaphoreType.DMA,
        ],
    )
    return f(mp, ipk)


# speedup vs baseline: 1.1108x; 1.1108x over previous
import jax
import jax.numpy as jnp
from jax import lax
from jax.experimental import pallas as pl
from jax.experimental.pallas import tpu as pltpu
from jax.experimental.pallas import tpu_sc as plsc

R = 128
N = 10000
E = 320000
NC = 2
NS = 16
L = 16
RB = 8
EH = E // NC
CH = 1280
NCH = EH // CH

_HI = -65536


def _body(mp_hbm, idx_hbm, out_hbm,
          rows, ipkA, ipkB, obA, obB,
          isemA, isemB, osemA, osemB):
    grp = lax.axis_index("s")
    half = lax.axis_index("c")
    r0 = pl.multiple_of(grp * RB, 8)
    ebase = pl.multiple_of(half * EH, 128)

    for r in range(RB):
        pltpu.sync_copy(mp_hbm.at[pl.ds((r0 + r) * N, N)],
                        rows.at[pl.ds(r * N, N)])

    def compute_chunk(ipk, ob):
        @plsc.parallel_loop(0, CH, step=L, unroll=2)
        def vec_body(s):
            iv = ipk[pl.ds(s, L)]
            ia = iv & 0xFFFF
            ib = lax.shift_right_logical(iv, 16)
            for r in range(RB):
                row = rows.at[pl.ds(r * N, N)]
                w1 = plsc.load_gather(row, [ia])
                w2 = plsc.load_gather(row, [ib])
                m1 = plsc.bitcast(w1, jnp.float32)
                p1 = plsc.bitcast(lax.shift_left(w1, 16), jnp.float32)
                m2 = plsc.bitcast(w2, jnp.float32)
                p2 = plsc.bitcast(lax.shift_left(w2, 16), jnp.float32)
                t1 = m1 * p2
                t2 = m2 * p1
                ob[r, pl.ds(s, L)] = (t1 * t2) / (t1 + t2)

    def phase(k, ipkX, isemX, obX, osemX, ipkY, isemY, pre_c, pre_ok):
        @pl.when(pre_ok)
        def _():
            pltpu.async_copy(idx_hbm.at[pl.ds(ebase + pre_c * CH, CH)],
                             ipkY, isemY)

        pltpu.make_async_copy(idx_hbm.at[pl.ds(0, CH)], ipkX, isemX).wait()

        @pl.when(k >= 2)
        def _():
            pltpu.make_async_copy(
                obX, out_hbm.at[pl.ds(0, RB), pl.ds(0, CH)], osemX).wait()

        compute_chunk(ipkX, obX)

        pltpu.async_copy(
            obX,
            out_hbm.at[pl.ds(r0, RB), pl.ds(ebase + k * CH, CH)], osemX)

    pltpu.async_copy(idx_hbm.at[pl.ds(ebase, CH)], ipkA, isemA)

    def pair_body(j, carry):
        k = 2 * j
        phase(k, ipkA, isemA, obA, osemA, ipkB, isemB, k + 1, k + 1 < NCH)
        phase(k + 1, ipkB, isemB, obB, osemB, ipkA, isemA, k + 2, k + 2 < NCH)
        return carry

    lax.fori_loop(0, NCH // 2, pair_body, 0)
    phase(NCH - 1, ipkA, isemA, obA, osemA, ipkB, isemB, 0, False)

    pltpu.make_async_copy(
        obA, out_hbm.at[pl.ds(0, RB), pl.ds(0, CH)], osemA).wait()
    pltpu.make_async_copy(
        obB, out_hbm.at[pl.ds(0, RB), pl.ds(0, CH)], osemB).wait()


def kernel(m, polar, indices):
    mb = lax.bitcast_convert_type(m, jnp.uint32)
    pb = lax.bitcast_convert_type(polar, jnp.uint32)
    mp = lax.bitcast_convert_type(
        (mb & jnp.uint32(0xFFFF0000)) | (pb >> 16), jnp.int32).reshape(-1)
    ipk = indices[0] | (indices[1] << 16)

    mesh = plsc.VectorSubcoreMesh(core_axis_name="c", subcore_axis_name="s")
    f = pl.kernel(
        _body,
        out_type=jax.ShapeDtypeStruct((R, E), jnp.float32),
        mesh=mesh,
        compiler_params=pltpu.CompilerParams(needs_layout_passes=False),
        scratch_types=[
            pltpu.VMEM((RB * N,), jnp.int32),
            pltpu.VMEM((CH,), jnp.int32),
            pltpu.VMEM((CH,), jnp.int32),
            pltpu.VMEM((RB, CH), jnp.float32),
            pltpu.VMEM((RB, CH), jnp.float32),
            pltpu.SemaphoreType.DMA,
            pltpu.SemaphoreType.DMA,
            pltpu.SemaphoreType.DMA,
            pltpu.SemaphoreType.DMA,
        ],
    )
    return f(mp, ipk)


# speedup vs baseline: 1.1115x; 1.0007x over previous
import jax
import jax.numpy as jnp
from jax import lax
from jax.experimental import pallas as pl
from jax.experimental.pallas import tpu as pltpu
from jax.experimental.pallas import tpu_sc as plsc

R = 128
N = 10000
E = 320000
NC = 2
NS = 16
L = 16
RB = 8
EH = E // NC
CH = 1280
NCH = EH // CH

_HI = -65536


def _body(mp_hbm, idx_hbm, out_hbm,
          rows, ipkA, ipkB, obA, obB,
          isemA, isemB, osemA, osemB):
    grp = lax.axis_index("s")
    half = lax.axis_index("c")
    r0 = pl.multiple_of(grp * RB, 8)
    ebase = pl.multiple_of(half * EH, 128)

    for r in range(RB):
        pltpu.sync_copy(mp_hbm.at[pl.ds((r0 + r) * N, N)],
                        rows.at[pl.ds(r * N, N)])

    def compute_chunk(ipk, ob):
        @plsc.parallel_loop(0, CH, step=L, unroll=2)
        def vec_body(s):
            iv = ipk[pl.ds(s, L)]
            ia = iv & 0xFFFF
            ib = lax.shift_right_logical(iv, 16)
            for r in range(RB):
                row = rows.at[pl.ds(r * N, N)]
                w1 = plsc.load_gather(row, [ia])
                w2 = plsc.load_gather(row, [ib])
                m1 = plsc.bitcast(w1, jnp.float32)
                p1 = plsc.bitcast(lax.shift_left(w1, 16), jnp.float32)
                m2 = plsc.bitcast(w2, jnp.float32)
                p2 = plsc.bitcast(lax.shift_left(w2, 16), jnp.float32)
                t1 = m1 * p2
                t2 = m2 * p1
                ob[r, pl.ds(s, L)] = (t1 * t2) / (t1 + t2)

    def phase(k, ipkX, isemX, obX, osemX, ipkY, isemY, pre_c, pre_ok):
        @pl.when(pre_ok)
        def _():
            pltpu.async_copy(idx_hbm.at[pl.ds(ebase + pre_c * CH, CH)],
                             ipkY, isemY)

        pltpu.make_async_copy(idx_hbm.at[pl.ds(0, CH)], ipkX, isemX).wait()

        @pl.when(k >= 2)
        def _():
            pltpu.make_async_copy(
                obX, out_hbm.at[pl.ds(0, RB), pl.ds(0, CH)], osemX).wait()

        compute_chunk(ipkX, obX)

        pltpu.async_copy(
            obX,
            out_hbm.at[pl.ds(r0, RB), pl.ds(ebase + k * CH, CH)], osemX)

    pltpu.async_copy(idx_hbm.at[pl.ds(ebase, CH)], ipkA, isemA)

    def pair_body(j, carry):
        k = 2 * j
        phase(k, ipkA, isemA, obA, osemA, ipkB, isemB, k + 1, k + 1 < NCH)
        phase(k + 1, ipkB, isemB, obB, osemB, ipkA, isemA, k + 2, k + 2 < NCH)
        return carry

    lax.fori_loop(0, NCH // 2, pair_body, 0)
    phase(NCH - 1, ipkA, isemA, obA, osemA, ipkB, isemB, 0, False)

    pltpu.make_async_copy(
        obA, out_hbm.at[pl.ds(0, RB), pl.ds(0, CH)], osemA).wait()
    pltpu.make_async_copy(
        obB, out_hbm.at[pl.ds(0, RB), pl.ds(0, CH)], osemB).wait()


def kernel(m, polar, indices):
    mb = lax.bitcast_convert_type(
        m.astype(jnp.bfloat16), jnp.uint16).astype(jnp.uint32)
    pb = lax.bitcast_convert_type(
        polar.astype(jnp.bfloat16), jnp.uint16).astype(jnp.uint32)
    mp = lax.bitcast_convert_type((mb << 16) | pb, jnp.int32).reshape(-1)
    ipk = indices[0] | (indices[1] << 16)

    mesh = plsc.VectorSubcoreMesh(core_axis_name="c", subcore_axis_name="s")
    f = pl.kernel(
        _body,
        out_type=jax.ShapeDtypeStruct((R, E), jnp.float32),
        mesh=mesh,
        compiler_params=pltpu.CompilerParams(needs_layout_passes=False),
        scratch_types=[
            pltpu.VMEM((RB * N,), jnp.int32),
            pltpu.VMEM((CH,), jnp.int32),
            pltpu.VMEM((CH,), jnp.int32),
            pltpu.VMEM((RB, CH), jnp.float32),
            pltpu.VMEM((RB, CH), jnp.float32),
            pltpu.SemaphoreType.DMA,
            pltpu.SemaphoreType.DMA,
            pltpu.SemaphoreType.DMA,
            pltpu.SemaphoreType.DMA,
        ],
    )
    return f(mp, ipk)
